# baseline (speedup 1.0000x reference)
import jax
import jax.numpy as jnp
from jax.experimental import pallas as pl
from jax.experimental.pallas import tpu as pltpu

SLOPE = 0.2
HEADS = 4
NEG = -9e15

BR = 256
BO = 512

_NT = (((1,), (1,)), ((), ()))


def _prep_body(adj_ref, o_ref):
    o_ref[...] = jnp.where(adj_ref[...] > 0.0, 0.0, NEG).astype(jnp.bfloat16)


def _layer_body(bias_ref, x_ref, w_ref, asrc_ref, adst_ref, o_ref,
                hb_scr, f1a_scr, f1b_scr, f2r_scr, f2s_scr, hm_scr):
    r = pl.program_id(0)

    @pl.when(r == 0)
    def _():
        for hid in range(HEADS):
            h = jnp.dot(x_ref[...], w_ref[hid],
                        preferred_element_type=jnp.float32)
            f1 = jax.lax.dot_general(h, asrc_ref[hid][None], _NT,
                                     preferred_element_type=jnp.float32)
            f2c = jax.lax.dot_general(h, adst_ref[hid][None], _NT,
                                      preferred_element_type=jnp.float32)
            f2r = jax.lax.dot_general(adst_ref[hid][None], h, _NT,
                                      preferred_element_type=jnp.float32)
            z = f1 + jnp.max(f2c, axis=0, keepdims=True)
            c = jnp.maximum(z, z * SLOPE)
            hb_scr[hid] = h.astype(jnp.bfloat16)
            f1a_scr[hid] = f1 - c
            f1b_scr[hid] = f1 * SLOPE - c
            f2r_scr[hid] = f2r
            f2s_scr[hid] = f2r * SLOPE
            hm_scr[hid] = jnp.mean(h, axis=0, keepdims=True)

    rows = pl.ds(r * BR, BR)
    biasf = bias_ref[...].astype(jnp.float32)
    xblk = x_ref[rows, :]
    acc = None
    for hid in range(HEADS):
        e = jnp.maximum(f1a_scr[hid, rows, :] + f2r_scr[hid],
                        f1b_scr[hid, rows, :] + f2s_scr[hid])
        p = jnp.exp(e + biasf)
        s = jnp.sum(p, axis=1, keepdims=True)
        out = jnp.dot(p.astype(jnp.bfloat16), hb_scr[hid],
                      preferred_element_type=jnp.float32)
        bad = s == 0.0
        out = jnp.where(bad, hm_scr[hid], out / jnp.where(bad, 1.0, s))
        out = out + xblk
        out = jnp.where(out > 0.0, out, jnp.exp(out) - 1.0)
        acc = out if acc is None else acc + out
    o_ref[...] = acc * (1.0 / HEADS)


def _outer_body(a_ref, b_ref, o_ref):
    o_ref[...] = jax.lax.dot_general(a_ref[...].astype(jnp.bfloat16),
                                     b_ref[...].astype(jnp.bfloat16), _NT,
                                     preferred_element_type=jnp.float32)


def _gat_layer(xin, bias, W, a_src, a_dst, interpret=False):
    N, D = xin.shape
    nr = N // BR
    return pl.pallas_call(
        _layer_body,
        grid=(nr,),
        in_specs=[
            pl.BlockSpec((BR, N), lambda r: (r, 0)),
            pl.BlockSpec((N, D), lambda r: (0, 0)),
            pl.BlockSpec((HEADS, D, D), lambda r: (0, 0, 0)),
            pl.BlockSpec((HEADS, D), lambda r: (0, 0)),
            pl.BlockSpec((HEADS, D), lambda r: (0, 0)),
        ],
        out_specs=pl.BlockSpec((BR, D), lambda r: (r, 0)),
        out_shape=jax.ShapeDtypeStruct((N, D), jnp.float32),
        scratch_shapes=[
            pltpu.VMEM((HEADS, N, D), jnp.bfloat16),
            pltpu.VMEM((HEADS, N, 1), jnp.float32),
            pltpu.VMEM((HEADS, N, 1), jnp.float32),
            pltpu.VMEM((HEADS, 1, N), jnp.float32),
            pltpu.VMEM((HEADS, 1, N), jnp.float32),
            pltpu.VMEM((HEADS, 1, D), jnp.float32),
        ],
        interpret=interpret,
    )(bias, xin, W, a_src, a_dst)


def kernel(x, adj, W, a_src, a_dst, interpret=False):
    N, D = x.shape

    bias = pl.pallas_call(
        _prep_body,
        grid=(N // BR,),
        in_specs=[pl.BlockSpec((BR, N), lambda r: (r, 0))],
        out_specs=pl.BlockSpec((BR, N), lambda r: (r, 0)),
        out_shape=jax.ShapeDtypeStruct((N, N), jnp.bfloat16),
        interpret=interpret,
    )(adj)

    m = _gat_layer(x, bias, W, a_src, a_dst, interpret)
    m = _gat_layer(m, bias, W, a_src, a_dst, interpret)
    m = _gat_layer(m, bias, W, a_src, a_dst, interpret)

    nb = N // BO
    ret = pl.pallas_call(
        _outer_body,
        grid=(nb, nb),
        in_specs=[
            pl.BlockSpec((BO, D), lambda i, j: (i, 0)),
            pl.BlockSpec((BO, D), lambda i, j: (j, 0)),
        ],
        out_specs=pl.BlockSpec((BO, BO), lambda i, j: (i, j)),
        out_shape=jax.ShapeDtypeStruct((N, N), jnp.float32),
        interpret=interpret,
    )(m, m)
    return ret


# speedup vs baseline: 1.0011x; 1.0011x over previous
import jax
import jax.numpy as jnp
from jax.experimental import pallas as pl
from jax.experimental.pallas import tpu as pltpu

SLOPE = 0.2
HEADS = 4
NEG = -9e15

BR = 256
BO = 512

_NT = (((1,), (1,)), ((), ()))


def _prep_body(adj_ref, o_ref):
    o_ref[...] = jnp.where(adj_ref[...] > 0.0, 0.0, NEG).astype(jnp.bfloat16)


def _layer_body(bias_ref, x_ref, w_ref, asrc_ref, adst_ref, o_ref,
                hb_scr, f1a_scr, f1b_scr, f2r_scr, f2s_scr, hm_scr):
    r = pl.program_id(0)

    @pl.when(r == 0)
    def _():
        for hid in range(HEADS):
            h = jnp.dot(x_ref[...], w_ref[hid],
                        preferred_element_type=jnp.float32)
            f1 = jax.lax.dot_general(h, asrc_ref[hid][None], _NT,
                                     preferred_element_type=jnp.float32)
            f2c = jax.lax.dot_general(h, adst_ref[hid][None], _NT,
                                      preferred_element_type=jnp.float32)
            f2r = jax.lax.dot_general(adst_ref[hid][None], h, _NT,
                                      preferred_element_type=jnp.float32)
            z = f1 + jnp.max(f2c, axis=0, keepdims=True)
            c = jnp.maximum(z, z * SLOPE)
            hb_scr[hid] = h.astype(jnp.bfloat16)
            f1a_scr[hid] = f1 - c
            f1b_scr[hid] = f1 * SLOPE - c
            f2r_scr[hid] = f2r
            f2s_scr[hid] = f2r * SLOPE
            hm_scr[hid] = jnp.mean(h, axis=0, keepdims=True)

    rows = pl.ds(r * BR, BR)
    biasf = bias_ref[...].astype(jnp.float32)
    xblk = x_ref[rows, :]
    acc = None
    for hid in range(HEADS):
        e = jnp.maximum(f1a_scr[hid, rows, :] + f2r_scr[hid],
                        f1b_scr[hid, rows, :] + f2s_scr[hid])
        p = jnp.exp(e + biasf)
        s = jnp.sum(p, axis=1, keepdims=True)
        out = jnp.dot(p.astype(jnp.bfloat16), hb_scr[hid],
                      preferred_element_type=jnp.float32)
        bad = s == 0.0
        out = jnp.where(bad, hm_scr[hid], out / jnp.where(bad, 1.0, s))
        out = out + xblk
        out = jnp.where(out > 0.0, out, jnp.exp(out) - 1.0)
        acc = out if acc is None else acc + out
    o_ref[...] = acc * (1.0 / HEADS)


def _outer_body(a_ref, b_ref, o_ref):
    o_ref[...] = jax.lax.dot_general(a_ref[...], b_ref[...], _NT,
                                     preferred_element_type=jnp.float32)


def _gat_layer(xin, bias, W, a_src, a_dst, interpret=False):
    N, D = xin.shape
    nr = N // BR
    return pl.pallas_call(
        _layer_body,
        grid=(nr,),
        in_specs=[
            pl.BlockSpec((BR, N), lambda r: (r, 0)),
            pl.BlockSpec((N, D), lambda r: (0, 0)),
            pl.BlockSpec((HEADS, D, D), lambda r: (0, 0, 0)),
            pl.BlockSpec((HEADS, D), lambda r: (0, 0)),
            pl.BlockSpec((HEADS, D), lambda r: (0, 0)),
        ],
        out_specs=pl.BlockSpec((BR, D), lambda r: (r, 0)),
        out_shape=jax.ShapeDtypeStruct((N, D), jnp.float32),
        scratch_shapes=[
            pltpu.VMEM((HEADS, N, D), jnp.bfloat16),
            pltpu.VMEM((HEADS, N, 1), jnp.float32),
            pltpu.VMEM((HEADS, N, 1), jnp.float32),
            pltpu.VMEM((HEADS, 1, N), jnp.float32),
            pltpu.VMEM((HEADS, 1, N), jnp.float32),
            pltpu.VMEM((HEADS, 1, D), jnp.float32),
        ],
        interpret=interpret,
    )(bias, xin, W, a_src, a_dst)


def kernel(x, adj, W, a_src, a_dst, interpret=False):
    N, D = x.shape

    bias = pl.pallas_call(
        _prep_body,
        grid=(N // BR,),
        in_specs=[pl.BlockSpec((BR, N), lambda r: (r, 0))],
        out_specs=pl.BlockSpec((BR, N), lambda r: (r, 0)),
        out_shape=jax.ShapeDtypeStruct((N, N), jnp.bfloat16),
        interpret=interpret,
    )(adj)

    m = _gat_layer(x, bias, W, a_src, a_dst, interpret)
    m = _gat_layer(m, bias, W, a_src, a_dst, interpret)
    m = _gat_layer(m, bias, W, a_src, a_dst, interpret)

    nb = N // BO
    ret = pl.pallas_call(
        _outer_body,
        grid=(nb, nb),
        in_specs=[
            pl.BlockSpec((BO, D), lambda i, j: (i, 0)),
            pl.BlockSpec((BO, D), lambda i, j: (j, 0)),
        ],
        out_specs=pl.BlockSpec((BO, BO), lambda i, j: (i, j)),
        out_shape=jax.ShapeDtypeStruct((N, N), jnp.float32),
        interpret=interpret,
    )(m, m)
    return ret
